# Initial kernel scaffold; baseline (speedup 1.0000x reference)
import jax
import jax.numpy as jnp
from jax.experimental import pallas as pl


def kernel(x, edge_index, batch, pre_W, pre_b, bn_pre_g, bn_pre_b, conv1_W, conv1_b, bn1_g, bn1_b, bnp1_g, bnp1_b, conv2_W, conv2_b, bn2_g, bn2_b, bnp2_g, bnp2_b, post_W, post_b):
    raise NotImplementedError("write your pallas kernel here")


# speedup vs baseline: 8.7671x; 8.7671x over previous
import functools

import jax
import jax.numpy as jnp
from jax import lax
from jax.experimental import pallas as pl
from jax.experimental.pallas import tpu as pltpu
from jax.experimental.pallas import tpu_sc as plsc

N_NODES = 10000
N_PAD = 10240
E_EDGES = 320000
E_PAD = 327680
D_FEAT = 128
N_GRAPHS = 64
N_CLS = 10

NC = 2
NS = 16
NW = NC * NS
EPT = E_PAD // NW
EB = 128
NB = EPT // EB
NPT = N_PAD // NS

_sc_mesh = lambda: plsc.VectorSubcoreMesh(
    core_axis_name="c", subcore_axis_name="s", num_cores=NC, num_subcores=NS)


def _deg_body(dst_hbm, out_hbm, dstv, onesv, zbuf, acc):
    c = lax.axis_index("c")
    s = lax.axis_index("s")
    w = c * NS + s
    for i in range(EB // 16):
        onesv[pl.ds(i * 16, 16)] = jnp.ones((16,), jnp.float32)
    for i in range(NPT // 16):
        zbuf[pl.ds(i * 16, 16)] = jnp.zeros((16,), jnp.float32)
    pltpu.sync_copy(zbuf, acc.at[pl.ds(s * NPT, NPT)])
    plsc.subcore_barrier()

    def body(j, carry):
        base = w * EPT + j * EB
        pltpu.sync_copy(dst_hbm.at[pl.ds(base, EB)], dstv)
        pltpu.sync_copy(onesv, acc.at[dstv], add=True)
        return carry

    lax.fori_loop(0, NB, body, 0)
    plsc.subcore_barrier()
    pltpu.sync_copy(acc.at[pl.ds(s * NPT, NPT)],
                    out_hbm.at[pl.ds(c * N_PAD + s * NPT, NPT)])


def _sc_degree(dst_flat):
    k = pl.kernel(
        _deg_body,
        out_type=jax.ShapeDtypeStruct((NC * N_PAD,), jnp.float32),
        mesh=_sc_mesh(),
        scratch_types=[
            pltpu.VMEM((EB,), jnp.int32),
            pltpu.VMEM((EB,), jnp.float32),
            pltpu.VMEM((NPT,), jnp.float32),
            pltpu.VMEM_SHARED((N_PAD,), jnp.float32),
        ],
    )
    return k(dst_flat)


def _conv_body(h_hbm, src_hbm, dst_hbm, out_hbm, srcv, dstv, rows, zbuf, acc, sem):
    c = lax.axis_index("c")
    s = lax.axis_index("s")
    w = c * NS + s
    for i in range(16):
        for kk in range(D_FEAT // 16):
            zbuf[i, pl.ds(kk * 16, 16)] = jnp.zeros((16,), jnp.float32)

    def zb(i, carry):
        pltpu.sync_copy(zbuf, acc.at[pl.ds(s * NPT + i * 16, 16)])
        return carry

    lax.fori_loop(0, NPT // 16, zb, 0)
    plsc.subcore_barrier()

    pltpu.sync_copy(src_hbm.at[pl.ds(w * NB, NB)], srcv)
    pltpu.sync_copy(dst_hbm.at[pl.ds(w * NB, NB)], dstv)

    def body(j, carry):
        pltpu.async_copy(h_hbm.at[srcv.at[j]], rows, sem).wait()
        pltpu.sync_copy(rows, acc.at[dstv.at[j]], add=True)
        return carry

    lax.fori_loop(0, NB, body, 0)
    plsc.subcore_barrier()
    pltpu.sync_copy(acc.at[pl.ds(s * NPT, NPT)],
                    out_hbm.at[pl.ds(c * N_PAD + s * NPT, NPT)])


def _sc_conv(h, src2d, dst2d):
    k = pl.kernel(
        _conv_body,
        out_type=jax.ShapeDtypeStruct((NC * N_PAD, D_FEAT), jnp.float32),
        mesh=_sc_mesh(),
        scratch_types=[
            pltpu.VMEM((NB, EB), jnp.int32),
            pltpu.VMEM((NB, EB), jnp.int32),
            pltpu.VMEM((EB, D_FEAT), jnp.float32),
            pltpu.VMEM((16, D_FEAT), jnp.float32),
            pltpu.VMEM_SHARED((N_PAD, D_FEAT), jnp.float32),
            pltpu.SemaphoreType.DMA,
        ],
    )
    return k(h, src2d, dst2d)


def _row_mask(t):
    rows = lax.broadcasted_iota(jnp.int32, (N_PAD, 1), 0)
    return jnp.where(rows < N_NODES, t, 0.0)


def _bn(t, g, b, eps=1e-5):
    sm = jnp.sum(t, axis=0, keepdims=True)
    sq = jnp.sum(t * t, axis=0, keepdims=True)
    m = sm / N_NODES
    v = sq / N_NODES - m * m
    return (t - m) * lax.rsqrt(v + eps) * g + b


def _tc_pre_body(x_ref, w_ref, b_ref, g_ref, bb_ref, o_ref):
    h = jnp.dot(x_ref[...], w_ref[...], preferred_element_type=jnp.float32)
    h = h + b_ref[...]
    h = _row_mask(h)
    o_ref[...] = _bn(h, g_ref[...], bb_ref[...])


def _tc_pre(x_pad, pre_W, pre_b, g, b):
    return pl.pallas_call(
        _tc_pre_body,
        out_shape=jax.ShapeDtypeStruct((N_PAD, D_FEAT), jnp.float32),
    )(x_pad, pre_W, pre_b.reshape(1, -1), g.reshape(1, -1), b.reshape(1, -1))


def _tc_scale_body(deg_ref, h_ref, w_ref, dinv_ref, hp_ref):
    deg = deg_ref[0:N_PAD, :] + deg_ref[N_PAD:2 * N_PAD, :] + 1.0
    dinv = lax.rsqrt(deg)
    dinv_ref[...] = dinv
    hw = jnp.dot(h_ref[...], w_ref[...], preferred_element_type=jnp.float32)
    hp_ref[...] = dinv * hw


def _tc_scale(deg_parts, h, conv_W):
    return pl.pallas_call(
        _tc_scale_body,
        out_shape=(
            jax.ShapeDtypeStruct((N_PAD, 1), jnp.float32),
            jax.ShapeDtypeStruct((N_PAD, D_FEAT), jnp.float32),
        ),
    )(deg_parts.reshape(2 * N_PAD, 1), h, conv_W)


def _tc_mid_body(acc_ref, hp_ref, dinv_ref, cb_ref, g1_ref, b1_ref, g2_ref,
                 b2_ref, w_ref, hp2_ref):
    acc = acc_ref[0:N_PAD, :] + acc_ref[N_PAD:2 * N_PAD, :]
    t = dinv_ref[...] * (acc + hp_ref[...]) + cb_ref[...]
    t = jnp.maximum(t, 0.0)
    t = _row_mask(t)
    t = _bn(t, g1_ref[...], b1_ref[...])
    t = _row_mask(t)
    t = _bn(t, g2_ref[...], b2_ref[...])
    hw = jnp.dot(t, w_ref[...], preferred_element_type=jnp.float32)
    hp2_ref[...] = dinv_ref[...] * hw


def _tc_mid(acc, hp, dinv, conv_b, g1, b1, g2, b2, next_W):
    return pl.pallas_call(
        _tc_mid_body,
        out_shape=jax.ShapeDtypeStruct((N_PAD, D_FEAT), jnp.float32),
    )(acc, hp, dinv, conv_b.reshape(1, -1), g1.reshape(1, -1),
      b1.reshape(1, -1), g2.reshape(1, -1), b2.reshape(1, -1), next_W)


def _tc_final_body(acc_ref, hp_ref, dinv_ref, cb_ref, g1_ref, b1_ref, g2_ref,
                   b2_ref, batch_ref, pw_ref, pb_ref, y_ref):
    acc = acc_ref[0:N_PAD, :] + acc_ref[N_PAD:2 * N_PAD, :]
    t = dinv_ref[...] * (acc + hp_ref[...]) + cb_ref[...]
    t = jnp.maximum(t, 0.0)
    t = _row_mask(t)
    t = _bn(t, g1_ref[...], b1_ref[...])
    t = _row_mask(t)
    t = _bn(t, g2_ref[...], b2_ref[...])
    gids = lax.broadcasted_iota(jnp.int32, (N_GRAPHS, N_PAD), 0)
    onehot = jnp.where(gids == batch_ref[...], 1.0, 0.0)
    cnt = jnp.sum(onehot, axis=1, keepdims=True)
    pooled = jnp.dot(onehot, t, preferred_element_type=jnp.float32)
    pooled = pooled / jnp.maximum(cnt, 1.0)
    y = jnp.dot(pooled, pw_ref[...], preferred_element_type=jnp.float32)
    y = y + pb_ref[...]
    mx = jnp.max(y, axis=1, keepdims=True)
    e = jnp.exp(y - mx)
    lse = mx + jnp.log(jnp.sum(e, axis=1, keepdims=True))
    y_ref[...] = y - lse


def _tc_final(acc, hp, dinv, conv_b, g1, b1, g2, b2, batch_pad, post_W, post_b):
    return pl.pallas_call(
        _tc_final_body,
        out_shape=jax.ShapeDtypeStruct((N_GRAPHS, N_CLS), jnp.float32),
    )(acc, hp, dinv, conv_b.reshape(1, -1), g1.reshape(1, -1),
      b1.reshape(1, -1), g2.reshape(1, -1), b2.reshape(1, -1),
      batch_pad.reshape(1, N_PAD), post_W, post_b.reshape(1, -1))


@jax.jit
def kernel(x, edge_index, batch, pre_W, pre_b, bn_pre_g, bn_pre_b, conv1_W,
           conv1_b, bn1_g, bn1_b, bnp1_g, bnp1_b, conv2_W, conv2_b, bn2_g,
           bn2_b, bnp2_g, bnp2_b, post_W, post_b):
    x_pad = jnp.zeros((N_PAD, D_FEAT), jnp.float32).at[:N_NODES].set(x)
    pad_e = E_PAD - E_EDGES
    src_flat = jnp.concatenate([edge_index[0],
                                jnp.zeros((pad_e,), jnp.int32)])
    dst_flat = jnp.concatenate([edge_index[1],
                                jnp.full((pad_e,), N_NODES, jnp.int32)])
    src2d = src_flat.reshape(NW * NB, EB)
    dst2d = dst_flat.reshape(NW * NB, EB)
    batch_pad = jnp.concatenate(
        [batch, jnp.full((N_PAD - N_NODES,), N_GRAPHS, jnp.int32)])

    deg_parts = _sc_degree(dst_flat)
    h = _tc_pre(x_pad, pre_W, pre_b, bn_pre_g, bn_pre_b)
    dinv, hp1 = _tc_scale(deg_parts, h, conv1_W)
    acc1 = _sc_conv(hp1, src2d, dst2d)
    hp2 = _tc_mid(acc1, hp1, dinv, conv1_b, bn1_g, bn1_b,
                  bnp1_g, bnp1_b, conv2_W)
    acc2 = _sc_conv(hp2, src2d, dst2d)
    y = _tc_final(acc2, hp2, dinv, conv2_b, bn2_g, bn2_b,
                  bnp2_g, bnp2_b, batch_pad, post_W, post_b)
    return (y, 0.0)


# speedup vs baseline: 9.2426x; 1.0542x over previous
import functools

import jax
import jax.numpy as jnp
from jax import lax
from jax.experimental import pallas as pl
from jax.experimental.pallas import tpu as pltpu
from jax.experimental.pallas import tpu_sc as plsc

N_NODES = 10000
N_PAD = 10240
E_EDGES = 320000
E_PAD = 327680
D_FEAT = 128
N_GRAPHS = 64
N_CLS = 10

NC = 2
NS = 16
NW = NC * NS
EPT = E_PAD // NW
EB = 128
NB = EPT // EB
EBC = 128
NBC = EPT // EBC
NB2 = NBC // 2
NPT = N_PAD // NS

_sc_mesh = lambda: plsc.VectorSubcoreMesh(
    core_axis_name="c", subcore_axis_name="s", num_cores=NC, num_subcores=NS)


def _deg_body(dst_hbm, out_hbm, dstv, onesv, zbuf, acc):
    c = lax.axis_index("c")
    s = lax.axis_index("s")
    w = c * NS + s
    for i in range(EB // 16):
        onesv[pl.ds(i * 16, 16)] = jnp.ones((16,), jnp.float32)
    for i in range(NPT // 16):
        zbuf[pl.ds(i * 16, 16)] = jnp.zeros((16,), jnp.float32)
    pltpu.sync_copy(zbuf, acc.at[pl.ds(s * NPT, NPT)])
    plsc.subcore_barrier()

    def body(j, carry):
        base = w * EPT + j * EB
        pltpu.sync_copy(dst_hbm.at[pl.ds(base, EB)], dstv)
        pltpu.sync_copy(onesv, acc.at[dstv], add=True)
        return carry

    lax.fori_loop(0, NB, body, 0)
    plsc.subcore_barrier()
    pltpu.sync_copy(acc.at[pl.ds(s * NPT, NPT)],
                    out_hbm.at[pl.ds(c * N_PAD + s * NPT, NPT)])


def _sc_degree(dst_flat):
    k = pl.kernel(
        _deg_body,
        out_type=jax.ShapeDtypeStruct((NC * N_PAD,), jnp.float32),
        mesh=_sc_mesh(),
        scratch_types=[
            pltpu.VMEM((EB,), jnp.int32),
            pltpu.VMEM((EB,), jnp.float32),
            pltpu.VMEM((NPT,), jnp.float32),
            pltpu.VMEM_SHARED((N_PAD,), jnp.float32),
        ],
    )
    return k(dst_flat)


def _conv_body(h_hbm, src_hbm, dst_hbm, out_hbm, srcv, dstv, rows0, rows1,
               acc, gs0, gs1, ss0, ss1):
    c = lax.axis_index("c")
    s = lax.axis_index("s")
    w = c * NS + s

    def zr(i, carry):
        for kk in range(D_FEAT // 16):
            rows0[i, pl.ds(kk * 16, 16)] = jnp.zeros((16,), jnp.float32)
        return carry

    lax.fori_loop(0, EBC, zr, 0)

    def zb(i, carry):
        pltpu.sync_copy(rows0, acc.at[pl.ds(s * NPT + i * EBC, EBC)])
        return carry

    lax.fori_loop(0, NPT // EBC, zb, 0)
    plsc.subcore_barrier()

    for phase in range(2):
        base = w * NBC + phase * NB2
        pltpu.sync_copy(src_hbm.at[pl.ds(base, NB2)], srcv)
        pltpu.sync_copy(dst_hbm.at[pl.ds(base, NB2)], dstv)
        pltpu.async_copy(h_hbm.at[srcv.at[0]], rows0, gs0)

        def body(i, carry):
            j0 = 2 * i
            j1 = j0 + 1
            pltpu.make_async_copy(h_hbm.at[srcv.at[j0]], rows0, gs0).wait()
            pltpu.async_copy(rows0, acc.at[dstv.at[j0]], ss0, add=True)

            @pl.when(i > 0)
            def _():
                pltpu.make_async_copy(rows1, acc.at[dstv.at[0]], ss1).wait()

            pltpu.async_copy(h_hbm.at[srcv.at[j1]], rows1, gs1)
            pltpu.make_async_copy(h_hbm.at[srcv.at[j1]], rows1, gs1).wait()
            pltpu.async_copy(rows1, acc.at[dstv.at[j1]], ss1, add=True)
            pltpu.make_async_copy(rows0, acc.at[dstv.at[0]], ss0).wait()
            jn = jnp.minimum(j0 + 2, NB2 - 1)
            pltpu.async_copy(h_hbm.at[srcv.at[jn]], rows0, gs0)
            return carry

        lax.fori_loop(0, NB2 // 2, body, 0)
        pltpu.make_async_copy(rows1, acc.at[dstv.at[0]], ss1).wait()
        pltpu.make_async_copy(h_hbm.at[srcv.at[0]], rows0, gs0).wait()

    plsc.subcore_barrier()
    pltpu.sync_copy(acc.at[pl.ds(s * NPT, NPT)],
                    out_hbm.at[pl.ds(c * N_PAD + s * NPT, NPT)])


def _sc_conv(h, src2d, dst2d):
    k = pl.kernel(
        _conv_body,
        out_type=jax.ShapeDtypeStruct((NC * N_PAD, D_FEAT), jnp.float32),
        mesh=_sc_mesh(),
        scratch_types=[
            pltpu.VMEM((NB2, EBC), jnp.int32),
            pltpu.VMEM((NB2, EBC), jnp.int32),
            pltpu.VMEM((EBC, D_FEAT), jnp.float32),
            pltpu.VMEM((EBC, D_FEAT), jnp.float32),
            pltpu.VMEM_SHARED((N_PAD, D_FEAT), jnp.float32),
            pltpu.SemaphoreType.DMA,
            pltpu.SemaphoreType.DMA,
            pltpu.SemaphoreType.DMA,
            pltpu.SemaphoreType.DMA,
        ],
    )
    return k(h, src2d, dst2d)


def _row_mask(t):
    rows = lax.broadcasted_iota(jnp.int32, (N_PAD, 1), 0)
    return jnp.where(rows < N_NODES, t, 0.0)


def _bn(t, g, b, eps=1e-5):
    sm = jnp.sum(t, axis=0, keepdims=True)
    sq = jnp.sum(t * t, axis=0, keepdims=True)
    m = sm / N_NODES
    v = sq / N_NODES - m * m
    return (t - m) * lax.rsqrt(v + eps) * g + b


def _tc_pre_body(x_ref, w_ref, b_ref, g_ref, bb_ref, o_ref):
    h = jnp.dot(x_ref[...], w_ref[...], preferred_element_type=jnp.float32)
    h = h + b_ref[...]
    h = _row_mask(h)
    o_ref[...] = _bn(h, g_ref[...], bb_ref[...])


def _tc_pre(x_pad, pre_W, pre_b, g, b):
    return pl.pallas_call(
        _tc_pre_body,
        out_shape=jax.ShapeDtypeStruct((N_PAD, D_FEAT), jnp.float32),
    )(x_pad, pre_W, pre_b.reshape(1, -1), g.reshape(1, -1), b.reshape(1, -1))


def _tc_scale_body(deg_ref, h_ref, w_ref, dinv_ref, hp_ref):
    deg = deg_ref[0:N_PAD, :] + deg_ref[N_PAD:2 * N_PAD, :] + 1.0
    dinv = lax.rsqrt(deg)
    dinv_ref[...] = dinv
    hw = jnp.dot(h_ref[...], w_ref[...], preferred_element_type=jnp.float32)
    hp_ref[...] = dinv * hw


def _tc_scale(deg_parts, h, conv_W):
    return pl.pallas_call(
        _tc_scale_body,
        out_shape=(
            jax.ShapeDtypeStruct((N_PAD, 1), jnp.float32),
            jax.ShapeDtypeStruct((N_PAD, D_FEAT), jnp.float32),
        ),
    )(deg_parts.reshape(2 * N_PAD, 1), h, conv_W)


def _tc_mid_body(acc_ref, hp_ref, dinv_ref, cb_ref, g1_ref, b1_ref, g2_ref,
                 b2_ref, w_ref, hp2_ref):
    acc = acc_ref[0:N_PAD, :] + acc_ref[N_PAD:2 * N_PAD, :]
    t = dinv_ref[...] * (acc + hp_ref[...]) + cb_ref[...]
    t = jnp.maximum(t, 0.0)
    t = _row_mask(t)
    t = _bn(t, g1_ref[...], b1_ref[...])
    t = _row_mask(t)
    t = _bn(t, g2_ref[...], b2_ref[...])
    hw = jnp.dot(t, w_ref[...], preferred_element_type=jnp.float32)
    hp2_ref[...] = dinv_ref[...] * hw


def _tc_mid(acc, hp, dinv, conv_b, g1, b1, g2, b2, next_W):
    return pl.pallas_call(
        _tc_mid_body,
        out_shape=jax.ShapeDtypeStruct((N_PAD, D_FEAT), jnp.float32),
    )(acc, hp, dinv, conv_b.reshape(1, -1), g1.reshape(1, -1),
      b1.reshape(1, -1), g2.reshape(1, -1), b2.reshape(1, -1), next_W)


def _tc_final_body(acc_ref, hp_ref, dinv_ref, cb_ref, g1_ref, b1_ref, g2_ref,
                   b2_ref, batch_ref, pw_ref, pb_ref, y_ref):
    acc = acc_ref[0:N_PAD, :] + acc_ref[N_PAD:2 * N_PAD, :]
    t = dinv_ref[...] * (acc + hp_ref[...]) + cb_ref[...]
    t = jnp.maximum(t, 0.0)
    t = _row_mask(t)
    t = _bn(t, g1_ref[...], b1_ref[...])
    t = _row_mask(t)
    t = _bn(t, g2_ref[...], b2_ref[...])
    gids = lax.broadcasted_iota(jnp.int32, (N_GRAPHS, N_PAD), 0)
    onehot = jnp.where(gids == batch_ref[...], 1.0, 0.0)
    cnt = jnp.sum(onehot, axis=1, keepdims=True)
    pooled = jnp.dot(onehot, t, preferred_element_type=jnp.float32)
    pooled = pooled / jnp.maximum(cnt, 1.0)
    y = jnp.dot(pooled, pw_ref[...], preferred_element_type=jnp.float32)
    y = y + pb_ref[...]
    mx = jnp.max(y, axis=1, keepdims=True)
    e = jnp.exp(y - mx)
    lse = mx + jnp.log(jnp.sum(e, axis=1, keepdims=True))
    y_ref[...] = y - lse


def _tc_final(acc, hp, dinv, conv_b, g1, b1, g2, b2, batch_pad, post_W, post_b):
    return pl.pallas_call(
        _tc_final_body,
        out_shape=jax.ShapeDtypeStruct((N_GRAPHS, N_CLS), jnp.float32),
    )(acc, hp, dinv, conv_b.reshape(1, -1), g1.reshape(1, -1),
      b1.reshape(1, -1), g2.reshape(1, -1), b2.reshape(1, -1),
      batch_pad.reshape(1, N_PAD), post_W, post_b.reshape(1, -1))


@jax.jit
def kernel(x, edge_index, batch, pre_W, pre_b, bn_pre_g, bn_pre_b, conv1_W,
           conv1_b, bn1_g, bn1_b, bnp1_g, bnp1_b, conv2_W, conv2_b, bn2_g,
           bn2_b, bnp2_g, bnp2_b, post_W, post_b):
    x_pad = jnp.zeros((N_PAD, D_FEAT), jnp.float32).at[:N_NODES].set(x)
    pad_e = E_PAD - E_EDGES
    src_flat = jnp.concatenate([edge_index[0],
                                jnp.zeros((pad_e,), jnp.int32)])
    dst_flat = jnp.concatenate([edge_index[1],
                                jnp.full((pad_e,), N_NODES, jnp.int32)])
    src2d = src_flat.reshape(NW * NBC, EBC)
    dst2d = dst_flat.reshape(NW * NBC, EBC)
    batch_pad = jnp.concatenate(
        [batch, jnp.full((N_PAD - N_NODES,), N_GRAPHS, jnp.int32)])

    deg_parts = _sc_degree(dst_flat)
    h = _tc_pre(x_pad, pre_W, pre_b, bn_pre_g, bn_pre_b)
    dinv, hp1 = _tc_scale(deg_parts, h, conv1_W)
    acc1 = _sc_conv(hp1, src2d, dst2d)
    hp2 = _tc_mid(acc1, hp1, dinv, conv1_b, bn1_g, bn1_b,
                  bnp1_g, bnp1_b, conv2_W)
    acc2 = _sc_conv(hp2, src2d, dst2d)
    y = _tc_final(acc2, hp2, dinv, conv2_b, bn2_g, bn2_b,
                  bnp2_g, bnp2_b, batch_pad, post_W, post_b)
    return (y, 0.0)
